# baseline (speedup 1.0000x reference)
import functools

import jax
import jax.numpy as jnp
import numpy as np
from jax import lax
from jax.experimental import pallas as pl
from jax.experimental.pallas import tpu as pltpu
from jax.experimental.pallas import tpu_sc as plsc

_N = 10000
_E = 160000
_D = 128
_HGC = 64
_HID = 16
_NS = 5
_NSC = 2
_NSUB = 16
_NW = _NSC * _NSUB
_CH = 128
_NCH = 40
_RPS = 624
_FC = 104
_NF = 6
_TAILB = _NSUB * _RPS
_TAILN = _N - _TAILB
_INVC = float(1.0 / np.sqrt(1.0 + 1e-5))

@functools.lru_cache(maxsize=None)
def _mesh():
    return plsc.VectorSubcoreMesh(
        core_axis_name="c", subcore_axis_name="s",
        num_cores=_NSC, num_subcores=_NSUB,
    )

def _z16():
    return jnp.zeros((16,), jnp.float32)


def _zero2d(ref, rows, cols):
    total = rows * cols
    nv = (total + 15) // 16
    lane = lax.iota(jnp.int32, 16)

    def body(i, _):
        flat = i * 16 + lane
        plsc.store_scatter(ref, [flat // cols, flat % cols], _z16(),
                           mask=flat < total)
        return 0

    lax.fori_loop(0, nv, body, 0)


_BE = 2048
_EPAD = _NW * _NCH * _CH
_BR = _BE // _CH


def _edge_mlp_body(x_ref, row_ref, col_ref, w1_ref, b1_ref, w2_ref, b2_ref,
                   w3_ref, b3_ref, ew_ref, ewm_ref):
    x = x_ref[...]

    def tower(xh):
        h = jnp.dot(xh, w1_ref[...], preferred_element_type=jnp.float32) + b1_ref[...]
        h = jnp.maximum(h, 0.0) * _INVC
        h = jnp.dot(h, w2_ref[...], preferred_element_type=jnp.float32) + b2_ref[...]
        h = jnp.maximum(h, 0.0) * _INVC
        return jnp.dot(h, w3_ref[...], preferred_element_type=jnp.float32) + b3_ref[...]

    h1 = tower(x[:, :16])
    h2 = tower(x[:, 16:])
    n1 = jnp.maximum(jnp.sqrt(jnp.sum(h1 * h1, axis=1)), 1e-8)
    n2 = jnp.maximum(jnp.sqrt(jnp.sum(h2 * h2, axis=1)), 1e-8)
    cos = jnp.sum(h1 * h2, axis=1) / (n1 * n2)
    ew = ((cos + 1.0) * 0.5).reshape(_BR, _CH)
    row2 = row_ref[...]
    valid = row2 >= 0
    ew_ref[...] = jnp.where(valid, ew, 0.0)
    ewm_ref[...] = jnp.where(valid & (row2 != col_ref[...]), ew, 0.0)


def _edge_weights(edgenet_input, row, col, w1t, b1, w2t, b2, w3t, b3):
    xp = jnp.pad(edgenet_input, ((0, _EPAD - _E), (0, 0)))
    rowm = jnp.pad(row, (0, _EPAD - _E), constant_values=-1).reshape(-1, _CH)
    colp = jnp.pad(col, (0, _EPAD - _E)).reshape(-1, _CH)
    grid = _EPAD // _BE
    return pl.pallas_call(
        _edge_mlp_body,
        grid=(grid,),
        in_specs=[
            pl.BlockSpec((_BE, 32), lambda i: (i, 0)),
            pl.BlockSpec((_BR, _CH), lambda i: (i, 0)),
            pl.BlockSpec((_BR, _CH), lambda i: (i, 0)),
            pl.BlockSpec((16, 512), lambda i: (0, 0)),
            pl.BlockSpec((512,), lambda i: (0,)),
            pl.BlockSpec((512, 256), lambda i: (0, 0)),
            pl.BlockSpec((256,), lambda i: (0,)),
            pl.BlockSpec((256, 128), lambda i: (0, 0)),
            pl.BlockSpec((128,), lambda i: (0,)),
        ],
        out_specs=[
            pl.BlockSpec((_BR, _CH), lambda i: (i, 0)),
            pl.BlockSpec((_BR, _CH), lambda i: (i, 0)),
        ],
        out_shape=[
            jax.ShapeDtypeStruct((_EPAD // _CH, _CH), jnp.float32),
            jax.ShapeDtypeStruct((_EPAD // _CH, _CH), jnp.float32),
        ],
    )(xp, rowm, colp, w1t, b1, w2t, b2, w3t, b3)


@functools.lru_cache(maxsize=None)
def _deg_kernel_build():
    return functools.partial(
        pl.kernel,
        out_type=jax.ShapeDtypeStruct((_NSC, _N, 8), jnp.float32),
        mesh=_mesh(),
        compiler_params=pltpu.CompilerParams(needs_layout_passes=False, use_tc_tiling_on_sc=False),
        scratch_types=[
            pltpu.VMEM((_NCH, _CH), jnp.int32),
            pltpu.VMEM((_NCH, _CH), jnp.float32),
            pltpu.VMEM((_CH, 8), jnp.float32),
            pltpu.VMEM((_FC, 8), jnp.float32),
            pltpu.VMEM_SHARED((_N, 8), jnp.float32),
            pltpu.SemaphoreType.DMA,
        ],
    )(_deg_body)


def _edge_sweep(src_hbm, acc_s, idxr_v, idxc_v, wn_v, gbufs, sbufs,
                gsems, ssems, width, nch):
    nbuf = len(gbufs)

    def fire_gather(j, b):
        pltpu.async_copy(src_hbm.at[idxr_v.at[j]], gbufs[b], gsems[b])

    def wait_gather(j, b):
        pltpu.make_async_copy(src_hbm.at[idxr_v.at[j]], gbufs[b],
                              gsems[b]).wait()

    def fire_scatter(j, b):
        pltpu.async_copy(sbufs[b], acc_s.at[idxc_v.at[j]], ssems[b], add=True)

    def wait_scatter(j, b):
        pltpu.make_async_copy(sbufs[b], acc_s.at[idxc_v.at[j]],
                              ssems[b]).wait()

    def scale(j, b):
        gbuf, sbuf = gbufs[b], sbufs[b]

        if width == 8:
            lane = lax.iota(jnp.int32, 16)
            p01 = lane >> 3
            cc = lane & 7

            def body8(k, carry):
                wadr, rc = carry
                w = plsc.load_gather(wn_v, [wadr])
                g = plsc.load_gather(gbuf, [rc, cc])
                plsc.store_scatter(sbuf, [rc, cc], g * w)
                return (wadr + 2, rc + 2)

            plsc.parallel_loop(
                0, _CH // 2, 1, unroll=8,
                carry=(jnp.full((16,), j * _CH, jnp.int32) + p01,
                       p01))(body8)
        else:
            def body(e, adr):
                w = plsc.load_gather(wn_v, [adr])
                for q in range(width // 16):
                    sl = pl.ds(q * 16, 16)
                    sbuf[e, sl] = gbuf[e, sl] * w
                return adr + 1

            plsc.parallel_loop(0, _CH, 1, unroll=8,
                               carry=jnp.full((16,), j * _CH, jnp.int32))(body)

    for b in range(nbuf):
        fire_gather(b, b)
    for b in range(nbuf):
        wait_gather(b, b)
        scale(b, b)
        fire_scatter(b, b)
        fire_gather(b + nbuf, b)

    def grp(g, _):
        for b in range(nbuf):
            j = g * nbuf + b
            wait_gather(j, b)
            wait_scatter(j - nbuf, b)
            scale(j, b)
            fire_scatter(j, b)

            @pl.when(j + nbuf < nch)
            def _(j=j, b=b):
                fire_gather(j + nbuf, b)

        return 0

    lax.fori_loop(1, nch // nbuf, grp, 0)
    for b in range(nbuf):
        wait_scatter(nch - nbuf + b, b)


def _rowchunks(s, fn):
    for k in range(_NF):
        fn(s * _RPS + k * _FC, _FC)

    @pl.when(s == _NSUB - 1)
    def _():
        fn(_TAILB, _TAILN)


def _deg_body(row_hbm, w_hbm, out_hbm, idx_v, w_v, buf_v, ob_v, acc_s, sem):
    c = lax.axis_index("c")
    s = lax.axis_index("s")
    wid = c * _NSUB + s
    pltpu.sync_copy(row_hbm.at[wid], idx_v)
    pltpu.sync_copy(w_hbm.at[wid], w_v)
    _zero2d(buf_v, _CH, 8)
    _zero2d(ob_v, _FC, 8)
    _rowchunks(s, lambda b, n: pltpu.sync_copy(
        ob_v.at[pl.ds(0, n)], acc_s.at[pl.ds(b, n)]))
    plsc.subcore_barrier()

    lane = lax.iota(jnp.int32, 16)
    zlane = jnp.zeros((16,), jnp.int32)

    def chunk(j, _):
        for i in range(_CH // 16):
            v = w_v[j, pl.ds(i * 16, 16)]
            plsc.store_scatter(buf_v, [lane + (i * 16), zlane], v)
        pltpu.sync_copy(buf_v, acc_s.at[idx_v.at[j]], add=True)
        return 0

    lax.fori_loop(0, _NCH, chunk, 0)
    plsc.subcore_barrier()

    def out_copy(b, n):
        pltpu.sync_copy(acc_s.at[pl.ds(b, n)], ob_v.at[pl.ds(0, n)])
        pltpu.sync_copy(ob_v.at[pl.ds(0, n)], out_hbm.at[c, pl.ds(b, n)])

    _rowchunks(s, out_copy)


def _prep_body(deg_ref, f_ref, lwt_ref, lb_ref, dinv_ref, hb0_ref):
    d = deg_ref[0, :, 0] + deg_ref[1, :, 0]
    dinv_ref[...] = jnp.where(d > 0, lax.rsqrt(jnp.maximum(d, 1e-30)), 0.0)
    hb0_ref[...] = (
        jnp.dot(f_ref[...], lwt_ref[...], preferred_element_type=jnp.float32)
        + lb_ref[...]
    )


def _node_prep(deg8, features, lwt, lb):
    return pl.pallas_call(
        _prep_body,
        out_shape=[
            jax.ShapeDtypeStruct((_N,), jnp.float32),
            jax.ShapeDtypeStruct((_N, _HID), jnp.float32),
        ],
    )(deg8, features, lwt, lb)


@functools.lru_cache(maxsize=None)
def _wnorm_kernel_build():
    return functools.partial(
        pl.kernel,
        out_type=jax.ShapeDtypeStruct((_NW, _NCH, _CH), jnp.float32),
        mesh=_mesh(),
        compiler_params=pltpu.CompilerParams(needs_layout_passes=False, use_tc_tiling_on_sc=False),
        scratch_types=[
            pltpu.VMEM((_NCH, _CH), jnp.int32),
            pltpu.VMEM((_NCH, _CH), jnp.int32),
            pltpu.VMEM((_NCH, _CH), jnp.float32),
            pltpu.VMEM((_N,), jnp.float32),
            pltpu.SemaphoreType.DMA,
        ],
    )(_wnorm_body)


def _wnorm_body(row_hbm, col_hbm, ew_hbm, dinv_hbm, out_hbm,
                idxr_v, idxc_v, w_v, dinv_v, sem):
    c = lax.axis_index("c")
    s = lax.axis_index("s")
    wid = c * _NSUB + s
    pltpu.sync_copy(row_hbm.at[wid], idxr_v)
    pltpu.sync_copy(col_hbm.at[wid], idxc_v)
    pltpu.sync_copy(ew_hbm.at[wid], w_v)
    pltpu.sync_copy(dinv_hbm, dinv_v)

    def chunk(j, _):
        for i in range(_CH // 16):
            sl = pl.ds(i * 16, 16)
            g1 = plsc.load_gather(dinv_v, [idxr_v[j, sl]])
            g2 = plsc.load_gather(dinv_v, [idxc_v[j, sl]])
            w_v[j, sl] = -(g1 * w_v[j, sl] * g2)
        return 0

    lax.fori_loop(0, _NCH, chunk, 0)
    pltpu.sync_copy(w_v, out_hbm.at[wid])


@functools.lru_cache(maxsize=None)
def _make_chain(width):
    @functools.partial(
        pl.kernel,
        out_type=[jax.ShapeDtypeStruct((2 * _N, width), jnp.float32)] * 3,
        mesh=_mesh(),
        compiler_params=pltpu.CompilerParams(needs_layout_passes=False, use_tc_tiling_on_sc=False),
        scratch_types=[
            pltpu.VMEM((2 * _NCH, _CH), jnp.int32),
            pltpu.VMEM((2 * _NCH, _CH), jnp.int32),
            pltpu.VMEM((2 * _NCH * _CH,), jnp.float32),
            pltpu.VMEM((_CH, width), jnp.float32),
            pltpu.VMEM((_CH, width), jnp.float32),
            pltpu.VMEM((_CH, width), jnp.float32),
            pltpu.VMEM((_CH, width), jnp.float32),
            pltpu.VMEM((_FC, width), jnp.float32),
            pltpu.VMEM((_FC, width), jnp.float32),
            pltpu.VMEM((_FC, width), jnp.float32),
            pltpu.VMEM_SHARED((_N, width), jnp.float32),
            pltpu.SemaphoreType.DMA,
            pltpu.SemaphoreType.DMA,
            pltpu.SemaphoreType.DMA,
            pltpu.SemaphoreType.DMA,
        ],
    )
    def chain(x_hbm, row_hbm, col_hbm, wn_hbm, tx1_hbm, tx2_hbm, tx3_hbm,
              idxr_v, idxc_v, wn_v, r0, r1, r2, r3, ab_v, pb_v, zb_v, acc_s,
              g0, g1, s0, s1):
        c = lax.axis_index("c")
        s = lax.axis_index("s")
        cn = c * _N
        pltpu.sync_copy(row_hbm.at[s], idxr_v.at[pl.ds(0, _NCH)])
        pltpu.sync_copy(row_hbm.at[s + _NSUB], idxr_v.at[pl.ds(_NCH, _NCH)])
        pltpu.sync_copy(col_hbm.at[s], idxc_v.at[pl.ds(0, _NCH)])
        pltpu.sync_copy(col_hbm.at[s + _NSUB], idxc_v.at[pl.ds(_NCH, _NCH)])
        pltpu.sync_copy(wn_hbm.at[s], wn_v.at[pl.ds(0, _NCH * _CH)])
        pltpu.sync_copy(wn_hbm.at[s + _NSUB],
                        wn_v.at[pl.ds(_NCH * _CH, _NCH * _CH)])

        cnv = jnp.full((16,), cn, jnp.int32)

        def adj(k, _):
            j = k // (_CH // 16)
            sl = pl.ds((k % (_CH // 16)) * 16, 16)
            idxr_v[j, sl] = idxr_v[j, sl] + cnv
            return 0

        lax.fori_loop(0, 2 * _NCH * (_CH // 16), adj, 0)

        def zb(k, _):
            r = k // (width // 16)
            sl = pl.ds((k % (width // 16)) * 16, 16)
            zb_v[r, sl] = _z16()
            return 0

        lax.fori_loop(0, _FC * (width // 16), zb, 0)
        _rowchunks(s, lambda b, n: pltpu.sync_copy(
            zb_v.at[pl.ds(0, n)], acc_s.at[pl.ds(b, n)]))
        plsc.subcore_barrier()

        def lhat(src_hbm):
            _edge_sweep(src_hbm, acc_s, idxr_v, idxc_v, wn_v,
                        (r0, r1), (r2, r3), (g0, g1), (s0, s1),
                        width, 2 * _NCH)
            plsc.subcore_barrier()

        def finish(dst_hbm, prev_hbm):
            def one(b, n):
                pltpu.sync_copy(acc_s.at[pl.ds(b, n)], ab_v.at[pl.ds(0, n)])
                if prev_hbm is not None:
                    pltpu.sync_copy(prev_hbm.at[pl.ds(cn + b, n)],
                                    pb_v.at[pl.ds(0, n)])

                    def rr(r, _):
                        for q in range(width // 16):
                            sl = pl.ds(q * 16, 16)
                            ab_v[r, sl] = ab_v[r, sl] * 2.0 - pb_v[r, sl]
                        return 0

                    lax.fori_loop(0, n, rr, 0)
                pltpu.sync_copy(ab_v.at[pl.ds(0, n)],
                                dst_hbm.at[pl.ds(cn + b, n)])
                pltpu.sync_copy(zb_v.at[pl.ds(0, n)], acc_s.at[pl.ds(b, n)])

            _rowchunks(s, one)
            plsc.subcore_barrier()

        lhat(x_hbm)
        finish(tx1_hbm, None)
        lhat(tx1_hbm)
        finish(tx2_hbm, x_hbm)
        lhat(tx2_hbm)
        finish(tx3_hbm, tx1_hbm)

    return chain


@functools.lru_cache(maxsize=None)
def _bys_kernel_build():
    return functools.partial(
        pl.kernel,
        out_type=[
            jax.ShapeDtypeStruct((_NS * _NS * 2 * _N, _HID // 2), jnp.float32),
            jax.ShapeDtypeStruct((2 * _N, _HID // 2), jnp.float32),
        ],
        mesh=_mesh(),
        compiler_params=pltpu.CompilerParams(needs_layout_passes=False, use_tc_tiling_on_sc=False),
        scratch_types=[
            pltpu.VMEM((2 * _NCH, _CH), jnp.int32),
            pltpu.VMEM((2 * _NCH, _CH), jnp.int32),
            pltpu.VMEM((2 * _NCH * _CH,), jnp.float32),
            pltpu.VMEM((_CH, _HID // 2), jnp.float32),
            pltpu.VMEM((_CH, _HID // 2), jnp.float32),
            pltpu.VMEM((_CH, _HID // 2), jnp.float32),
            pltpu.VMEM((_CH, _HID // 2), jnp.float32),
            pltpu.VMEM((_CH, _HID // 2), jnp.float32),
            pltpu.VMEM((_CH, _HID // 2), jnp.float32),
            pltpu.VMEM((_CH, _HID // 2), jnp.float32),
            pltpu.VMEM((_CH, _HID // 2), jnp.float32),
            pltpu.VMEM((_RPS, _HID // 2), jnp.float32),
            pltpu.VMEM((_TAILN, _HID // 2), jnp.float32),
            pltpu.VMEM_SHARED((_N, _HID // 2), jnp.float32),
            pltpu.SemaphoreType.DMA,
            pltpu.SemaphoreType.DMA,
            pltpu.SemaphoreType.DMA,
            pltpu.SemaphoreType.DMA,
            pltpu.SemaphoreType.DMA,
            pltpu.SemaphoreType.DMA,
            pltpu.SemaphoreType.DMA,
            pltpu.SemaphoreType.DMA,
        ],
    )(_bys_body)


def _bys_body(hb0_hbm, row_hbm, col_hbm, w_hbm, prior_hbm, samp_hbm, hbc_hbm,
              idxr_v, idxc_v, w_v, r0, r1, r2, r3, r4, r5, r6, r7,
              ab_v, at_v, acc_s, g0, g1, g2, g3, s0, s1, s2, s3):
    c = lax.axis_index("c")
    s = lax.axis_index("s")
    cn = c * _N
    pltpu.sync_copy(row_hbm.at[s], idxr_v.at[pl.ds(0, _NCH)])
    pltpu.sync_copy(row_hbm.at[s + _NSUB], idxr_v.at[pl.ds(_NCH, _NCH)])
    pltpu.sync_copy(col_hbm.at[s], idxc_v.at[pl.ds(0, _NCH)])
    pltpu.sync_copy(col_hbm.at[s + _NSUB], idxc_v.at[pl.ds(_NCH, _NCH)])
    pltpu.sync_copy(w_hbm.at[s], w_v.at[pl.ds(0, _NCH * _CH)])
    pltpu.sync_copy(w_hbm.at[s + _NSUB],
                    w_v.at[pl.ds(_NCH * _CH, _NCH * _CH)])

    cnv = jnp.full((16,), cn, jnp.int32)

    def adj(k, _):
        j = k // (_CH // 16)
        sl = pl.ds((k % (_CH // 16)) * 16, 16)
        idxr_v[j, sl] = idxr_v[j, sl] + cnv
        return 0

    lax.fori_loop(0, 2 * _NCH * (_CH // 16), adj, 0)

    def perchunk(fn):
        fn(s * _RPS, _RPS, ab_v)

        @pl.when(s == _NSUB - 1)
        def _():
            fn(_TAILB, _TAILN, at_v)

    def init(b, n, buf):
        pltpu.sync_copy(prior_hbm.at[pl.ds(cn + b, n)], acc_s.at[pl.ds(b, n)])
        pltpu.sync_copy(hb0_hbm.at[pl.ds(cn + b, n)], buf.at[pl.ds(0, n)])
        pltpu.sync_copy(buf.at[pl.ds(0, n)], hbc_hbm.at[pl.ds(cn + b, n)])

    perchunk(init)
    plsc.subcore_barrier()

    def step(su, _):
        _edge_sweep(hbc_hbm, acc_s, idxr_v, idxc_v, w_v,
                    (r0, r1, r2, r3), (r4, r5, r6, r7),
                    (g0, g1, g2, g3), (s0, s1, s2, s3),
                    _HID // 2, 2 * _NCH)
        plsc.subcore_barrier()

        def fin(b, n, buf):
            pltpu.sync_copy(acc_s.at[pl.ds(b, n)], buf.at[pl.ds(0, n)])
            pltpu.sync_copy(buf.at[pl.ds(0, n)], hbc_hbm.at[pl.ds(cn + b, n)])
            pltpu.sync_copy(buf.at[pl.ds(0, n)],
                            samp_hbm.at[pl.ds(su * 2 * _N + cn + b, n)])
            pltpu.sync_copy(
                prior_hbm.at[pl.ds(
                    jnp.minimum(su + 1, _NS * _NS - 1) * 2 * _N + cn + b, n)],
                acc_s.at[pl.ds(b, n)])

        perchunk(fin)
        plsc.subcore_barrier()
        return 0

    lax.fori_loop(0, _NS * _NS, step, 0)


def _chebmm_body(x_ref, w_ref, o_ref):
    o_ref[...] = jnp.maximum(
        jnp.dot(x_ref[...], w_ref[...], preferred_element_type=jnp.float32), 0.0
    )


def _chebmm(xcat, wcat_t):
    nb = 2000
    kdim = xcat.shape[1]
    return pl.pallas_call(
        _chebmm_body,
        grid=(_N // nb,),
        in_specs=[
            pl.BlockSpec((nb, kdim), lambda i: (i, 0)),
            pl.BlockSpec((kdim, _HGC), lambda i: (0, 0)),
        ],
        out_specs=pl.BlockSpec((nb, _HGC), lambda i: (i, 0)),
        out_shape=jax.ShapeDtypeStruct((_N, _HGC), jnp.float32),
    )(xcat, wcat_t)


def _final_body(jk_ref, hs_ref, w1_ref, b1_ref, w2_ref, b2_ref, wl_ref, bl_ref,
                l1_ref, l2_ref, pr_ref):
    t = jnp.dot(jk_ref[...], w1_ref[...], preferred_element_type=jnp.float32) + b1_ref[...]
    t = jnp.maximum(t, 0.0) * _INVC
    lg1 = jnp.dot(t, w2_ref[...], preferred_element_type=jnp.float32) + b2_ref[...]
    lg2 = jnp.dot(hs_ref[...], wl_ref[...], preferred_element_type=jnp.float32) + bl_ref[...]
    lg2 = jax.nn.sigmoid(lg2)
    lane = lax.broadcasted_iota(jnp.int32, lg1.shape, 1)
    mask = lane < 2

    def smax(z):
        zm = jnp.where(mask, z, -1e30)
        mx = jnp.max(zm, axis=1, keepdims=True)
        e = jnp.exp(zm - mx)
        return e / jnp.sum(e, axis=1, keepdims=True)

    l1 = smax(lg1)
    l2 = smax(lg2)
    l1_ref[...] = l1
    l2_ref[...] = l2
    pr_ref[...] = jnp.maximum(l1, l2)


def _final(jk, hs, w1t, b1, w2tp, b2p, wltp, blp):
    nb = 2000
    outs = pl.pallas_call(
        _final_body,
        grid=(_N // nb,),
        in_specs=[
            pl.BlockSpec((nb, 3 * _HGC), lambda i: (i, 0)),
            pl.BlockSpec((nb, _NS * _NS * _HID), lambda i: (i, 0)),
            pl.BlockSpec((3 * _HGC, 256), lambda i: (0, 0)),
            pl.BlockSpec((256,), lambda i: (0,)),
            pl.BlockSpec((256, 128), lambda i: (0, 0)),
            pl.BlockSpec((128,), lambda i: (0,)),
            pl.BlockSpec((_NS * _NS * _HID, 128), lambda i: (0, 0)),
            pl.BlockSpec((128,), lambda i: (0,)),
        ],
        out_specs=[pl.BlockSpec((nb, 128), lambda i: (i, 0))] * 3,
        out_shape=[jax.ShapeDtypeStruct((_N, 128), jnp.float32)] * 3,
    )(jk, hs, w1t, b1, w2tp, b2p, wltp, blp)
    return outs


def _pad_chunks(x, workers, nch):
    cap = workers * nch * _CH
    return jnp.pad(x, (0, cap - _E)).reshape(workers, nch, _CH)


def _colsplit(x):
    h = x.shape[1] // 2
    return jnp.concatenate([x[:, :h], x[:, h:]], axis=0)


def _coljoin(x):
    return jnp.concatenate([x[:_N], x[_N:]], axis=1)


def kernel(features, edge_index, edgenet_input, en_W1, en_b1, en_W2, en_b2,
           en_W3, en_b3, cheb0_W, cheb1_W, cheb2_W, m2_W1, m2_b1, m2_W2, m2_b2,
           L_W, L_b, hL_W, hL_b, prior_noise):
    row, col = edge_index[0], edge_index[1]

    ew2d, ewm2d = _edge_weights(edgenet_input, row, col, en_W1.T, en_b1,
                                en_W2.T, en_b2, en_W3.T, en_b3)

    row32 = _pad_chunks(row, _NW, _NCH)
    col32 = _pad_chunks(col, _NW, _NCH)
    ewm32 = ewm2d.reshape(_NW, _NCH, _CH)
    ew32 = ew2d.reshape(_NW, _NCH * _CH)

    deg8 = _deg_kernel_build()(row32, ewm32)
    dinv, hb0 = _node_prep(deg8, features, L_W.T, L_b)
    wn32 = _wnorm_kernel_build()(row32, col32, ewm32, dinv)
    wn32 = wn32.reshape(_NW, _NCH * _CH)

    x_full = features
    x_sc = _colsplit(features)
    h0 = None
    jk = None
    for li, W in enumerate((cheb0_W, cheb1_W, cheb2_W)):
        chain = _make_chain(64) if li == 0 else _make_chain(32)
        tx1, tx2, tx3 = chain(x_sc, row32, col32, wn32)
        xcat = jnp.concatenate(
            [x_full, _coljoin(tx1), _coljoin(tx2), _coljoin(tx3)], axis=1)
        wcat_t = jnp.concatenate([W[0], W[1], W[2], W[3]], axis=1).T
        h = _chebmm(xcat, wcat_t)
        if li == 0:
            h0 = h
            jk = h
        else:
            jk = jnp.concatenate([h0, h], axis=1)
            h0 = jk
        x_full = h
        x_sc = _colsplit(h)

    pr = prior_noise.reshape(_NS * _NS, _N, _HID)
    prior_cs = jnp.concatenate(
        [pr[:, :, :_HID // 2], pr[:, :, _HID // 2:]], axis=1
    ).reshape(_NS * _NS * 2 * _N, _HID // 2)
    hb0_cs = _colsplit(hb0)
    samples, _hbc = _bys_kernel_build()(hb0_cs, row32, col32, ew32, prior_cs)
    t = samples.reshape(_NS * _NS, 2, _N, _HID // 2)
    h_samples = jnp.concatenate([t[:, 0], t[:, 1]], axis=-1).reshape(
        _N, _NS * _NS * _HID)

    w2p = jnp.zeros((256, 128), jnp.float32).at[:, :2].set(m2_W2.T)
    b2p = jnp.zeros((128,), jnp.float32).at[:2].set(m2_b2)
    wlp = jnp.zeros((_NS * _NS * _HID, 128), jnp.float32).at[:, :2].set(hL_W.T)
    blp = jnp.zeros((128,), jnp.float32).at[:2].set(hL_b)
    l1p, l2p, prp = _final(jk, h_samples, m2_W1.T, m2_b1, w2p, b2p, wlp, blp)
    return (l1p[:, :2], l2p[:, :2], prp[:, :2])


# speedup vs baseline: 1.0520x; 1.0520x over previous
import functools

import jax
import jax.numpy as jnp
import numpy as np
from jax import lax
from jax.experimental import pallas as pl
from jax.experimental.pallas import tpu as pltpu
from jax.experimental.pallas import tpu_sc as plsc

_N = 10000
_E = 160000
_D = 128
_HGC = 64
_HID = 16
_NS = 5
_NSC = 2
_NSUB = 16
_NW = _NSC * _NSUB
_CH = 128
_NCH = 40
_RPS = 624
_FC = 104
_NF = 6
_TAILB = _NSUB * _RPS
_TAILN = _N - _TAILB
_INVC = float(1.0 / np.sqrt(1.0 + 1e-5))

@functools.lru_cache(maxsize=None)
def _mesh():
    return plsc.VectorSubcoreMesh(
        core_axis_name="c", subcore_axis_name="s",
        num_cores=_NSC, num_subcores=_NSUB,
    )

def _z16():
    return jnp.zeros((16,), jnp.float32)


def _zero2d(ref, rows, cols):
    total = rows * cols
    nv = (total + 15) // 16
    lane = lax.iota(jnp.int32, 16)

    def body(i, _):
        flat = i * 16 + lane
        plsc.store_scatter(ref, [flat // cols, flat % cols], _z16(),
                           mask=flat < total)
        return 0

    lax.fori_loop(0, nv, body, 0)


_BE = 2048
_EPAD = _NW * _NCH * _CH
_BR = _BE // _CH


def _edge_mlp_body(x_ref, row_ref, col_ref, w1_ref, b1_ref, w2_ref, b2_ref,
                   w3_ref, b3_ref, ew_ref, ewm_ref):
    x = x_ref[...]

    def tower(xh):
        h = jnp.dot(xh, w1_ref[...], preferred_element_type=jnp.float32) + b1_ref[...]
        h = jnp.maximum(h, 0.0) * _INVC
        h = jnp.dot(h, w2_ref[...], preferred_element_type=jnp.float32) + b2_ref[...]
        h = jnp.maximum(h, 0.0) * _INVC
        return jnp.dot(h, w3_ref[...], preferred_element_type=jnp.float32) + b3_ref[...]

    h1 = tower(x[:, :16])
    h2 = tower(x[:, 16:])
    n1 = jnp.maximum(jnp.sqrt(jnp.sum(h1 * h1, axis=1)), 1e-8)
    n2 = jnp.maximum(jnp.sqrt(jnp.sum(h2 * h2, axis=1)), 1e-8)
    cos = jnp.sum(h1 * h2, axis=1) / (n1 * n2)
    ew = ((cos + 1.0) * 0.5).reshape(_BR, _CH)
    row2 = row_ref[...]
    valid = row2 >= 0
    ew_ref[...] = jnp.where(valid, ew, 0.0)
    ewm_ref[...] = jnp.where(valid & (row2 != col_ref[...]), ew, 0.0)


def _edge_weights(edgenet_input, row, col, w1t, b1, w2t, b2, w3t, b3):
    xp = jnp.pad(edgenet_input, ((0, _EPAD - _E), (0, 0)))
    rowm = jnp.pad(row, (0, _EPAD - _E), constant_values=-1).reshape(-1, _CH)
    colp = jnp.pad(col, (0, _EPAD - _E)).reshape(-1, _CH)
    grid = _EPAD // _BE
    return pl.pallas_call(
        _edge_mlp_body,
        grid=(grid,),
        in_specs=[
            pl.BlockSpec((_BE, 32), lambda i: (i, 0)),
            pl.BlockSpec((_BR, _CH), lambda i: (i, 0)),
            pl.BlockSpec((_BR, _CH), lambda i: (i, 0)),
            pl.BlockSpec((16, 512), lambda i: (0, 0)),
            pl.BlockSpec((512,), lambda i: (0,)),
            pl.BlockSpec((512, 256), lambda i: (0, 0)),
            pl.BlockSpec((256,), lambda i: (0,)),
            pl.BlockSpec((256, 128), lambda i: (0, 0)),
            pl.BlockSpec((128,), lambda i: (0,)),
        ],
        out_specs=[
            pl.BlockSpec((_BR, _CH), lambda i: (i, 0)),
            pl.BlockSpec((_BR, _CH), lambda i: (i, 0)),
        ],
        out_shape=[
            jax.ShapeDtypeStruct((_EPAD // _CH, _CH), jnp.float32),
            jax.ShapeDtypeStruct((_EPAD // _CH, _CH), jnp.float32),
        ],
    )(xp, rowm, colp, w1t, b1, w2t, b2, w3t, b3)


@functools.lru_cache(maxsize=None)
def _deg_kernel_build():
    return functools.partial(
        pl.kernel,
        out_type=jax.ShapeDtypeStruct((_NSC, _N, 8), jnp.float32),
        mesh=_mesh(),
        compiler_params=pltpu.CompilerParams(needs_layout_passes=False, use_tc_tiling_on_sc=False),
        scratch_types=[
            pltpu.VMEM((_NCH, _CH), jnp.int32),
            pltpu.VMEM((_NCH, _CH), jnp.float32),
            pltpu.VMEM((_CH, 8), jnp.float32),
            pltpu.VMEM((_FC, 8), jnp.float32),
            pltpu.VMEM_SHARED((_N, 8), jnp.float32),
            pltpu.SemaphoreType.DMA,
        ],
    )(_deg_body)


def _edge_sweep(src_hbm, acc_s, idxr_v, idxc_v, wn_v, gbufs, sbufs,
                gsems, ssems, width, nch):
    nbuf = len(gbufs)

    def fire_gather(j, b):
        pltpu.async_copy(src_hbm.at[idxr_v.at[j]], gbufs[b], gsems[b])

    def wait_gather(j, b):
        pltpu.make_async_copy(src_hbm.at[idxr_v.at[j]], gbufs[b],
                              gsems[b]).wait()

    def fire_scatter(j, b):
        pltpu.async_copy(sbufs[b], acc_s.at[idxc_v.at[j]], ssems[b], add=True)

    def wait_scatter(j, b):
        pltpu.make_async_copy(sbufs[b], acc_s.at[idxc_v.at[j]],
                              ssems[b]).wait()

    def scale(j, b):
        gbuf, sbuf = gbufs[b], sbufs[b]

        if width == 8:
            lane = lax.iota(jnp.int32, 16)
            p01 = lane >> 3
            cc = lane & 7

            def body8(k, carry):
                wadr, rc = carry
                w = plsc.load_gather(wn_v, [wadr])
                g = plsc.load_gather(gbuf, [rc, cc])
                plsc.store_scatter(sbuf, [rc, cc], g * w)
                return (wadr + 2, rc + 2)

            plsc.parallel_loop(
                0, _CH // 2, 1, unroll=8,
                carry=(jnp.full((16,), j * _CH, jnp.int32) + p01,
                       p01))(body8)
        else:
            def body(e, adr):
                w = plsc.load_gather(wn_v, [adr])
                for q in range(width // 16):
                    sl = pl.ds(q * 16, 16)
                    sbuf[e, sl] = gbuf[e, sl] * w
                return adr + 1

            plsc.parallel_loop(0, _CH, 1, unroll=8,
                               carry=jnp.full((16,), j * _CH, jnp.int32))(body)

    for b in range(nbuf):
        fire_gather(b, b)
    for b in range(nbuf):
        wait_gather(b, b)
        scale(b, b)
        fire_scatter(b, b)
        fire_gather(b + nbuf, b)

    def grp(g, _):
        for b in range(nbuf):
            j = g * nbuf + b
            wait_gather(j, b)
            wait_scatter(j - nbuf, b)
            scale(j, b)
            fire_scatter(j, b)

            @pl.when(j + nbuf < nch)
            def _(j=j, b=b):
                fire_gather(j + nbuf, b)

        return 0

    lax.fori_loop(1, nch // nbuf, grp, 0)
    for b in range(nbuf):
        wait_scatter(nch - nbuf + b, b)


def _rowchunks(s, fn):
    for k in range(_NF):
        fn(s * _RPS + k * _FC, _FC)

    @pl.when(s == _NSUB - 1)
    def _():
        fn(_TAILB, _TAILN)


def _deg_body(row_hbm, w_hbm, out_hbm, idx_v, w_v, buf_v, ob_v, acc_s, sem):
    c = lax.axis_index("c")
    s = lax.axis_index("s")
    wid = c * _NSUB + s
    pltpu.sync_copy(row_hbm.at[wid], idx_v)
    pltpu.sync_copy(w_hbm.at[wid], w_v)
    _zero2d(buf_v, _CH, 8)
    _zero2d(ob_v, _FC, 8)
    _rowchunks(s, lambda b, n: pltpu.sync_copy(
        ob_v.at[pl.ds(0, n)], acc_s.at[pl.ds(b, n)]))
    plsc.subcore_barrier()

    lane = lax.iota(jnp.int32, 16)
    zlane = jnp.zeros((16,), jnp.int32)

    def chunk(j, _):
        for i in range(_CH // 16):
            v = w_v[j, pl.ds(i * 16, 16)]
            plsc.store_scatter(buf_v, [lane + (i * 16), zlane], v)
        pltpu.sync_copy(buf_v, acc_s.at[idx_v.at[j]], add=True)
        return 0

    lax.fori_loop(0, _NCH, chunk, 0)
    plsc.subcore_barrier()

    def out_copy(b, n):
        pltpu.sync_copy(acc_s.at[pl.ds(b, n)], ob_v.at[pl.ds(0, n)])
        pltpu.sync_copy(ob_v.at[pl.ds(0, n)], out_hbm.at[c, pl.ds(b, n)])

    _rowchunks(s, out_copy)


def _prep_body(deg_ref, f_ref, lwt_ref, lb_ref, dinv_ref, hb0_ref):
    d = deg_ref[0, :, 0] + deg_ref[1, :, 0]
    dinv_ref[...] = jnp.where(d > 0, lax.rsqrt(jnp.maximum(d, 1e-30)), 0.0)
    hb0_ref[...] = (
        jnp.dot(f_ref[...], lwt_ref[...], preferred_element_type=jnp.float32)
        + lb_ref[...]
    )


def _node_prep(deg8, features, lwt, lb):
    return pl.pallas_call(
        _prep_body,
        out_shape=[
            jax.ShapeDtypeStruct((_N,), jnp.float32),
            jax.ShapeDtypeStruct((_N, _HID), jnp.float32),
        ],
    )(deg8, features, lwt, lb)


@functools.lru_cache(maxsize=None)
def _wnorm_kernel_build():
    return functools.partial(
        pl.kernel,
        out_type=jax.ShapeDtypeStruct((_NW, _NCH, _CH), jnp.float32),
        mesh=_mesh(),
        compiler_params=pltpu.CompilerParams(needs_layout_passes=False, use_tc_tiling_on_sc=False),
        scratch_types=[
            pltpu.VMEM((_NCH, _CH), jnp.int32),
            pltpu.VMEM((_NCH, _CH), jnp.int32),
            pltpu.VMEM((_NCH, _CH), jnp.float32),
            pltpu.VMEM((_N,), jnp.float32),
            pltpu.SemaphoreType.DMA,
        ],
    )(_wnorm_body)


def _wnorm_body(row_hbm, col_hbm, ew_hbm, dinv_hbm, out_hbm,
                idxr_v, idxc_v, w_v, dinv_v, sem):
    c = lax.axis_index("c")
    s = lax.axis_index("s")
    wid = c * _NSUB + s
    pltpu.sync_copy(row_hbm.at[wid], idxr_v)
    pltpu.sync_copy(col_hbm.at[wid], idxc_v)
    pltpu.sync_copy(ew_hbm.at[wid], w_v)
    pltpu.sync_copy(dinv_hbm, dinv_v)

    def chunk(j, _):
        for i in range(_CH // 16):
            sl = pl.ds(i * 16, 16)
            g1 = plsc.load_gather(dinv_v, [idxr_v[j, sl]])
            g2 = plsc.load_gather(dinv_v, [idxc_v[j, sl]])
            w_v[j, sl] = -(g1 * w_v[j, sl] * g2)
        return 0

    lax.fori_loop(0, _NCH, chunk, 0)
    pltpu.sync_copy(w_v, out_hbm.at[wid])


@functools.lru_cache(maxsize=None)
def _make_chain(width):
    @functools.partial(
        pl.kernel,
        out_type=[jax.ShapeDtypeStruct((2 * _N, width), jnp.float32)] * 3,
        mesh=_mesh(),
        compiler_params=pltpu.CompilerParams(needs_layout_passes=False, use_tc_tiling_on_sc=False),
        scratch_types=[
            pltpu.VMEM((2 * _NCH, _CH), jnp.int32),
            pltpu.VMEM((2 * _NCH, _CH), jnp.int32),
            pltpu.VMEM((2 * _NCH * _CH,), jnp.float32),
            pltpu.VMEM((_CH, width), jnp.float32),
            pltpu.VMEM((_CH, width), jnp.float32),
            pltpu.VMEM((_CH, width), jnp.float32),
            pltpu.VMEM((_CH, width), jnp.float32),
            pltpu.VMEM((_FC, width), jnp.float32),
            pltpu.VMEM((_FC, width), jnp.float32),
            pltpu.VMEM((_FC, width), jnp.float32),
            pltpu.VMEM_SHARED((_N, width), jnp.float32),
            pltpu.SemaphoreType.DMA,
            pltpu.SemaphoreType.DMA,
            pltpu.SemaphoreType.DMA,
            pltpu.SemaphoreType.DMA,
        ],
    )
    def chain(x_hbm, row_hbm, col_hbm, wn_hbm, tx1_hbm, tx2_hbm, tx3_hbm,
              idxr_v, idxc_v, wn_v, r0, r1, r2, r3, ab_v, pb_v, zb_v, acc_s,
              g0, g1, s0, s1):
        c = lax.axis_index("c")
        s = lax.axis_index("s")
        cn = c * _N
        pltpu.sync_copy(row_hbm.at[s], idxr_v.at[pl.ds(0, _NCH)])
        pltpu.sync_copy(row_hbm.at[s + _NSUB], idxr_v.at[pl.ds(_NCH, _NCH)])
        pltpu.sync_copy(col_hbm.at[s], idxc_v.at[pl.ds(0, _NCH)])
        pltpu.sync_copy(col_hbm.at[s + _NSUB], idxc_v.at[pl.ds(_NCH, _NCH)])
        pltpu.sync_copy(wn_hbm.at[s], wn_v.at[pl.ds(0, _NCH * _CH)])
        pltpu.sync_copy(wn_hbm.at[s + _NSUB],
                        wn_v.at[pl.ds(_NCH * _CH, _NCH * _CH)])

        cnv = jnp.full((16,), cn, jnp.int32)

        def adj(k, _):
            j = k // (_CH // 16)
            sl = pl.ds((k % (_CH // 16)) * 16, 16)
            idxr_v[j, sl] = idxr_v[j, sl] + cnv
            return 0

        lax.fori_loop(0, 2 * _NCH * (_CH // 16), adj, 0)

        def zb(k, _):
            r = k // (width // 16)
            sl = pl.ds((k % (width // 16)) * 16, 16)
            zb_v[r, sl] = _z16()
            return 0

        lax.fori_loop(0, _FC * (width // 16), zb, 0)
        _rowchunks(s, lambda b, n: pltpu.sync_copy(
            zb_v.at[pl.ds(0, n)], acc_s.at[pl.ds(b, n)]))
        plsc.subcore_barrier()

        def lhat(src_hbm):
            _edge_sweep(src_hbm, acc_s, idxr_v, idxc_v, wn_v,
                        (r0, r1), (r2, r3), (g0, g1), (s0, s1),
                        width, 2 * _NCH)
            plsc.subcore_barrier()

        def finish(dst_hbm, prev_hbm):
            def one(b, n):
                pltpu.sync_copy(acc_s.at[pl.ds(b, n)], ab_v.at[pl.ds(0, n)])
                if prev_hbm is not None:
                    pltpu.sync_copy(prev_hbm.at[pl.ds(cn + b, n)],
                                    pb_v.at[pl.ds(0, n)])

                    def rr(r, _):
                        for q in range(width // 16):
                            sl = pl.ds(q * 16, 16)
                            ab_v[r, sl] = ab_v[r, sl] * 2.0 - pb_v[r, sl]
                        return 0

                    lax.fori_loop(0, n, rr, 0)
                pltpu.sync_copy(ab_v.at[pl.ds(0, n)],
                                dst_hbm.at[pl.ds(cn + b, n)])
                pltpu.sync_copy(zb_v.at[pl.ds(0, n)], acc_s.at[pl.ds(b, n)])

            _rowchunks(s, one)
            plsc.subcore_barrier()

        lhat(x_hbm)
        finish(tx1_hbm, None)
        lhat(tx1_hbm)
        finish(tx2_hbm, x_hbm)
        lhat(tx2_hbm)
        finish(tx3_hbm, tx1_hbm)

    return chain


@functools.lru_cache(maxsize=None)
def _bys_kernel_build():
    return functools.partial(
        pl.kernel,
        out_type=[
            jax.ShapeDtypeStruct((_NS * _NS * _N, _HID), jnp.float32),
            jax.ShapeDtypeStruct((_N, _HID), jnp.float32),
        ],
        mesh=_mesh(),
        compiler_params=pltpu.CompilerParams(needs_layout_passes=False, use_tc_tiling_on_sc=False),
        scratch_types=[
            pltpu.VMEM((2 * _NCH, _CH), jnp.int32),
            pltpu.VMEM((2 * _NCH, _CH), jnp.int32),
            pltpu.VMEM((2 * _NCH * _CH,), jnp.float32),
            pltpu.VMEM((_CH, _HID), jnp.float32),
            pltpu.VMEM((_CH, _HID), jnp.float32),
            pltpu.VMEM((_CH, _HID), jnp.float32),
            pltpu.VMEM((_CH, _HID), jnp.float32),
            pltpu.VMEM((_CH, _HID), jnp.float32),
            pltpu.VMEM((_CH, _HID), jnp.float32),
            pltpu.VMEM((_CH, _HID), jnp.float32),
            pltpu.VMEM((_CH, _HID), jnp.float32),
            pltpu.VMEM((_RPS, _HID), jnp.float32),
            pltpu.VMEM((_TAILN, _HID), jnp.float32),
            pltpu.VMEM_SHARED((_N, _HID), jnp.float32),
            pltpu.SemaphoreType.DMA,
            pltpu.SemaphoreType.DMA,
            pltpu.SemaphoreType.DMA,
            pltpu.SemaphoreType.DMA,
            pltpu.SemaphoreType.DMA,
            pltpu.SemaphoreType.DMA,
            pltpu.SemaphoreType.DMA,
            pltpu.SemaphoreType.DMA,
        ],
    )(_bys_body)


def _bys_body(hb0_hbm, row_hbm, col_hbm, w_hbm, prior_hbm, samp_hbm, hbc_hbm,
              idxr_v, idxc_v, w_v, r0, r1, r2, r3, r4, r5, r6, r7,
              ab_v, at_v, acc_s, g0, g1, g2, g3, s0, s1, s2, s3):
    c = lax.axis_index("c")
    s = lax.axis_index("s")

    @pl.when(c == 0)
    def _():
        pltpu.sync_copy(row_hbm.at[s], idxr_v)
        pltpu.sync_copy(col_hbm.at[s], idxc_v)
        pltpu.sync_copy(w_hbm.at[s], w_v)

        def perchunk(fn):
            fn(s * _RPS, _RPS, ab_v)

            @pl.when(s == _NSUB - 1)
            def _():
                fn(_TAILB, _TAILN, at_v)

        def init(b, n, buf):
            pltpu.sync_copy(prior_hbm.at[pl.ds(b, n)], acc_s.at[pl.ds(b, n)])
            pltpu.sync_copy(hb0_hbm.at[pl.ds(b, n)], buf.at[pl.ds(0, n)])
            pltpu.sync_copy(buf.at[pl.ds(0, n)], hbc_hbm.at[pl.ds(b, n)])

        perchunk(init)
        plsc.subcore_barrier()

        def step(su, _):
            _edge_sweep(hbc_hbm, acc_s, idxr_v, idxc_v, w_v,
                        (r0, r1, r2, r3), (r4, r5, r6, r7),
                        (g0, g1, g2, g3), (s0, s1, s2, s3),
                        _HID, 2 * _NCH)
            plsc.subcore_barrier()

            def fin(b, n, buf):
                pltpu.sync_copy(acc_s.at[pl.ds(b, n)], buf.at[pl.ds(0, n)])
                pltpu.sync_copy(buf.at[pl.ds(0, n)], hbc_hbm.at[pl.ds(b, n)])
                pltpu.sync_copy(buf.at[pl.ds(0, n)],
                                samp_hbm.at[pl.ds(su * _N + b, n)])
                pltpu.sync_copy(
                    prior_hbm.at[pl.ds(
                        jnp.minimum(su + 1, _NS * _NS - 1) * _N + b, n)],
                    acc_s.at[pl.ds(b, n)])

            perchunk(fin)
            plsc.subcore_barrier()
            return 0

        lax.fori_loop(0, _NS * _NS, step, 0)


def _chebmm_body(x_ref, w_ref, o_ref):
    o_ref[...] = jnp.maximum(
        jnp.dot(x_ref[...], w_ref[...], preferred_element_type=jnp.float32), 0.0
    )


def _chebmm(xcat, wcat_t):
    nb = 2000
    kdim = xcat.shape[1]
    return pl.pallas_call(
        _chebmm_body,
        grid=(_N // nb,),
        in_specs=[
            pl.BlockSpec((nb, kdim), lambda i: (i, 0)),
            pl.BlockSpec((kdim, _HGC), lambda i: (0, 0)),
        ],
        out_specs=pl.BlockSpec((nb, _HGC), lambda i: (i, 0)),
        out_shape=jax.ShapeDtypeStruct((_N, _HGC), jnp.float32),
    )(xcat, wcat_t)


def _final_body(jk_ref, hs_ref, w1_ref, b1_ref, w2_ref, b2_ref, wl_ref, bl_ref,
                l1_ref, l2_ref, pr_ref):
    t = jnp.dot(jk_ref[...], w1_ref[...], preferred_element_type=jnp.float32) + b1_ref[...]
    t = jnp.maximum(t, 0.0) * _INVC
    lg1 = jnp.dot(t, w2_ref[...], preferred_element_type=jnp.float32) + b2_ref[...]
    lg2 = jnp.dot(hs_ref[...], wl_ref[...], preferred_element_type=jnp.float32) + bl_ref[...]
    lg2 = jax.nn.sigmoid(lg2)
    lane = lax.broadcasted_iota(jnp.int32, lg1.shape, 1)
    mask = lane < 2

    def smax(z):
        zm = jnp.where(mask, z, -1e30)
        mx = jnp.max(zm, axis=1, keepdims=True)
        e = jnp.exp(zm - mx)
        return e / jnp.sum(e, axis=1, keepdims=True)

    l1 = smax(lg1)
    l2 = smax(lg2)
    l1_ref[...] = l1
    l2_ref[...] = l2
    pr_ref[...] = jnp.maximum(l1, l2)


def _final(jk, hs, w1t, b1, w2tp, b2p, wltp, blp):
    nb = 2000
    outs = pl.pallas_call(
        _final_body,
        grid=(_N // nb,),
        in_specs=[
            pl.BlockSpec((nb, 3 * _HGC), lambda i: (i, 0)),
            pl.BlockSpec((nb, _NS * _NS * _HID), lambda i: (i, 0)),
            pl.BlockSpec((3 * _HGC, 256), lambda i: (0, 0)),
            pl.BlockSpec((256,), lambda i: (0,)),
            pl.BlockSpec((256, 128), lambda i: (0, 0)),
            pl.BlockSpec((128,), lambda i: (0,)),
            pl.BlockSpec((_NS * _NS * _HID, 128), lambda i: (0, 0)),
            pl.BlockSpec((128,), lambda i: (0,)),
        ],
        out_specs=[pl.BlockSpec((nb, 128), lambda i: (i, 0))] * 3,
        out_shape=[jax.ShapeDtypeStruct((_N, 128), jnp.float32)] * 3,
    )(jk, hs, w1t, b1, w2tp, b2p, wltp, blp)
    return outs


def _pad_chunks(x, workers, nch):
    cap = workers * nch * _CH
    return jnp.pad(x, (0, cap - _E)).reshape(workers, nch, _CH)


def _colsplit(x):
    h = x.shape[1] // 2
    return jnp.concatenate([x[:, :h], x[:, h:]], axis=0)


def _coljoin(x):
    return jnp.concatenate([x[:_N], x[_N:]], axis=1)


def kernel(features, edge_index, edgenet_input, en_W1, en_b1, en_W2, en_b2,
           en_W3, en_b3, cheb0_W, cheb1_W, cheb2_W, m2_W1, m2_b1, m2_W2, m2_b2,
           L_W, L_b, hL_W, hL_b, prior_noise):
    row, col = edge_index[0], edge_index[1]

    ew2d, ewm2d = _edge_weights(edgenet_input, row, col, en_W1.T, en_b1,
                                en_W2.T, en_b2, en_W3.T, en_b3)

    row32 = _pad_chunks(row, _NW, _NCH)
    col32 = _pad_chunks(col, _NW, _NCH)
    ewm32 = ewm2d.reshape(_NW, _NCH, _CH)
    ew32 = ew2d.reshape(_NW, _NCH * _CH)

    deg8 = _deg_kernel_build()(row32, ewm32)
    dinv, hb0 = _node_prep(deg8, features, L_W.T, L_b)
    wn32 = _wnorm_kernel_build()(row32, col32, ewm32, dinv)
    wn32 = wn32.reshape(_NW, _NCH * _CH)

    x_full = features
    x_sc = _colsplit(features)
    h0 = None
    jk = None
    for li, W in enumerate((cheb0_W, cheb1_W, cheb2_W)):
        chain = _make_chain(64) if li == 0 else _make_chain(32)
        tx1, tx2, tx3 = chain(x_sc, row32, col32, wn32)
        xcat = jnp.concatenate(
            [x_full, _coljoin(tx1), _coljoin(tx2), _coljoin(tx3)], axis=1)
        wcat_t = jnp.concatenate([W[0], W[1], W[2], W[3]], axis=1).T
        h = _chebmm(xcat, wcat_t)
        if li == 0:
            h0 = h
            jk = h
        else:
            jk = jnp.concatenate([h0, h], axis=1)
            h0 = jk
        x_full = h
        x_sc = _colsplit(h)

    prior_flat = prior_noise.reshape(_NS * _NS * _N, _HID)
    row16 = row32.reshape(_NSUB, 2 * _NCH, _CH)
    col16 = col32.reshape(_NSUB, 2 * _NCH, _CH)
    ew16 = ew2d.reshape(_NSUB, 2 * _NCH * _CH)
    samples, _hbc = _bys_kernel_build()(hb0, row16, col16, ew16, prior_flat)
    h_samples = samples.reshape(_N, _NS * _NS * _HID)

    w2p = jnp.zeros((256, 128), jnp.float32).at[:, :2].set(m2_W2.T)
    b2p = jnp.zeros((128,), jnp.float32).at[:2].set(m2_b2)
    wlp = jnp.zeros((_NS * _NS * _HID, 128), jnp.float32).at[:, :2].set(hL_W.T)
    blp = jnp.zeros((128,), jnp.float32).at[:2].set(hL_b)
    l1p, l2p, prp = _final(jk, h_samples, m2_W1.T, m2_b1, w2p, b2p, wlp, blp)
    return (l1p[:, :2], l2p[:, :2], prp[:, :2])
